# Initial kernel scaffold; baseline (speedup 1.0000x reference)
import jax
import jax.numpy as jnp
from jax.experimental import pallas as pl


def kernel(x, edge_index, y, W1, b1, W2, b2):
    raise NotImplementedError("write your pallas kernel here")


# speedup vs baseline: 17.4514x; 17.4514x over previous
import functools

import jax
import jax.numpy as jnp
from jax import lax
from jax.experimental import pallas as pl
from jax.experimental.pallas import tpu as pltpu
from jax.experimental.pallas import tpu_sc as plsc

N = 50000
E = 1600000
F_IN = 128
HID = 64
C = 40
K = 10
ALPHA = 0.1

NC = 2
NS = 16
CR = C // NC
CF = 32
B = 128
CH = 8
OUTER = 98
NCHUNK = CH * OUTER
E_PAD = NS * NCHUNK * B
HALF = NCHUNK // 2
OUTER_D = OUTER // 2
NPAD = 50048
ZR = NPAD // NS

_mesh = plsc.VectorSubcoreMesh(core_axis_name="c", subcore_axis_name="s")


@functools.partial(
    pl.kernel,
    mesh=_mesh,
    out_type=jax.ShapeDtypeStruct((NC * NPAD,), jnp.float32),
    scratch_types=[
        pltpu.VMEM((CH, B), jnp.int32),
        pltpu.VMEM((B,), jnp.float32),
        pltpu.VMEM_SHARED((NPAD,), jnp.float32),
        pltpu.SemaphoreType.DMA,
    ],
    compiler_params=pltpu.CompilerParams(use_tc_tiling_on_sc=False),
)
def _deg_sc(cidx_hbm, zeros1_hbm, out_hbm, cidx_v, ones_v, acc, sem):
    c = lax.axis_index("c")
    s = lax.axis_index("s")
    pltpu.sync_copy(zeros1_hbm, acc.at[pl.ds(s * ZR, ZR)])
    for i in range(B // 16):
        ones_v[pl.ds(i * 16, 16)] = jnp.full((16,), 1.0, jnp.float32)
    plsc.subcore_barrier()

    def body(k, carry):
        pltpu.sync_copy(cidx_hbm.at[s, pl.ds(c * HALF + k * CH, CH)], cidx_v)
        for j in range(CH):
            pltpu.sync_copy(ones_v, acc.at[cidx_v.at[j]], add=True)
        return carry

    lax.fori_loop(0, OUTER_D, body, 0)
    plsc.subcore_barrier()
    pltpu.sync_copy(acc.at[pl.ds(s * ZR, ZR)],
                    out_hbm.at[pl.ds(c * NPAD + s * ZR, ZR)])


@functools.partial(
    pl.kernel,
    mesh=_mesh,
    out_type=jax.ShapeDtypeStruct((NC, NPAD, CF), jnp.float32),
    scratch_types=[
        pltpu.VMEM((CH, B), jnp.int32),
        pltpu.VMEM((CH, B), jnp.int32),
        pltpu.VMEM((B, CF), jnp.float32),
        pltpu.VMEM((B, CF), jnp.float32),
        pltpu.VMEM_SHARED((NPAD, CF), jnp.float32),
        pltpu.SemaphoreType.DMA,
        pltpu.SemaphoreType.DMA,
    ],
    compiler_params=pltpu.CompilerParams(use_tc_tiling_on_sc=False),
)
def _edge_sc(u2_hbm, ridx_hbm, cidx_hbm, zeros2_hbm, out_hbm,
             ridx_v, cidx_v, rows0, rows1, acc, sem0, sem1):
    c = lax.axis_index("c")
    s = lax.axis_index("s")
    pltpu.sync_copy(zeros2_hbm, acc.at[pl.ds(s * ZR, ZR)])
    plsc.subcore_barrier()
    rows = (rows0, rows1)
    sems = (sem0, sem1)
    uc = u2_hbm.at[c]

    def body(k, carry):
        pltpu.sync_copy(ridx_hbm.at[s, pl.ds(k * CH, CH)], ridx_v)
        pltpu.sync_copy(cidx_hbm.at[s, pl.ds(k * CH, CH)], cidx_v)
        copies = [None, None]
        copies[0] = pltpu.async_copy(uc.at[ridx_v.at[0]], rows[0], sems[0])
        for j in range(CH):
            nxt = (j + 1) % 2
            if j + 1 < CH:
                copies[nxt] = pltpu.async_copy(
                    uc.at[ridx_v.at[j + 1]], rows[nxt], sems[nxt])
            copies[j % 2].wait()
            pltpu.sync_copy(rows[j % 2], acc.at[cidx_v.at[j]], add=True)
        return carry

    lax.fori_loop(0, OUTER, body, 0)
    plsc.subcore_barrier()
    pltpu.sync_copy(acc.at[pl.ds(s * ZR, ZR)],
                    out_hbm.at[c, pl.ds(s * ZR, ZR)])


BN = 2000
GRID = N // BN


def _mlp_body(x_ref, w1_ref, b1_ref, w2_ref, b2_ref, deg_ref,
              h0_ref, u_ref, dinv_ref):
    h1 = jnp.maximum(
        jnp.dot(x_ref[...], w1_ref[...], preferred_element_type=jnp.float32)
        + b1_ref[...], 0.0)
    h0 = (jnp.dot(h1, w2_ref[...], preferred_element_type=jnp.float32)
          + b2_ref[...])
    deg = deg_ref[:, 0] + deg_ref[:, 1] + 1.0
    dinv = lax.rsqrt(deg)[:, None]
    h0_ref[...] = h0
    u = dinv * h0
    zpad = jnp.zeros((u.shape[0], CF - CR), jnp.float32)
    u_ref[0] = jnp.concatenate([u[:, :CR], zpad], axis=1)
    u_ref[1] = jnp.concatenate([u[:, CR:], zpad], axis=1)
    dinv_ref[...] = dinv


_mlp_tc = pl.pallas_call(
    _mlp_body,
    grid=(GRID,),
    in_specs=[
        pl.BlockSpec((BN, F_IN), lambda i: (i, 0)),
        pl.BlockSpec((F_IN, HID), lambda i: (0, 0)),
        pl.BlockSpec((1, HID), lambda i: (0, 0)),
        pl.BlockSpec((HID, C), lambda i: (0, 0)),
        pl.BlockSpec((1, C), lambda i: (0, 0)),
        pl.BlockSpec((BN, NC), lambda i: (i, 0)),
    ],
    out_specs=[
        pl.BlockSpec((BN, C), lambda i: (i, 0)),
        pl.BlockSpec((NC, BN, CF), lambda i: (0, i, 0)),
        pl.BlockSpec((BN, 1), lambda i: (i, 0)),
    ],
    out_shape=[
        jax.ShapeDtypeStruct((N, C), jnp.float32),
        jax.ShapeDtypeStruct((NC, N, CF), jnp.float32),
        jax.ShapeDtypeStruct((N, 1), jnp.float32),
    ],
)


def _combine_body(dinv_ref, h0_ref, u_ref, s_ref, out_ref):
    dinv = dinv_ref[...]
    for c in range(NC):
        agg = dinv * (s_ref[c][:, :CR] + u_ref[c][:, :CR])
        h0c = h0_ref[:, c * CR:(c + 1) * CR]
        nxt = dinv * ((1.0 - ALPHA) * agg + ALPHA * h0c)
        zpad = jnp.zeros((nxt.shape[0], CF - CR), jnp.float32)
        out_ref[c] = jnp.concatenate([nxt, zpad], axis=1)


def _final_body(dinv_ref, h0_ref, u_ref, s_ref, out_ref):
    dinv = dinv_ref[...]
    halves = []
    for c in range(NC):
        agg = dinv * (s_ref[c][:, :CR] + u_ref[c][:, :CR])
        h0c = h0_ref[:, c * CR:(c + 1) * CR]
        halves.append((1.0 - ALPHA) * agg + ALPHA * h0c)
    h = jnp.concatenate(halves, axis=1)
    m = jnp.max(h, axis=1, keepdims=True)
    lse = jnp.log(jnp.sum(jnp.exp(h - m), axis=1, keepdims=True)) + m
    out_ref[...] = h - lse


def _make_tc(body, out_spec, out_shape):
    return pl.pallas_call(
        body,
        grid=(GRID,),
        in_specs=[
            pl.BlockSpec((BN, 1), lambda i: (i, 0)),
            pl.BlockSpec((BN, C), lambda i: (i, 0)),
            pl.BlockSpec((NC, BN, CF), lambda i: (0, i, 0)),
            pl.BlockSpec((NC, BN, CF), lambda i: (0, i, 0)),
        ],
        out_specs=out_spec,
        out_shape=out_shape,
    )


_combine_tc = _make_tc(
    _combine_body,
    pl.BlockSpec((NC, BN, CF), lambda i: (0, i, 0)),
    jax.ShapeDtypeStruct((NC, N, CF), jnp.float32),
)
_final_tc = _make_tc(
    _final_body,
    pl.BlockSpec((BN, C), lambda i: (i, 0)),
    jax.ShapeDtypeStruct((N, C), jnp.float32),
)


def kernel(x, edge_index, y, W1, b1, W2, b2):
    del y
    row = edge_index[0].astype(jnp.int32)
    col = edge_index[1].astype(jnp.int32)
    pad = E_PAD - E
    ridx = jnp.concatenate([row, jnp.zeros((pad,), jnp.int32)])
    cidx = jnp.concatenate([col, jnp.full((pad,), N, jnp.int32)])
    ridx = ridx.reshape(NS, NCHUNK, B)
    cidx = cidx.reshape(NS, NCHUNK, B)
    zeros1 = jnp.zeros((ZR,), jnp.float32)
    zeros2 = jnp.zeros((ZR, CF), jnp.float32)

    deg_p = _deg_sc(cidx, zeros1)
    h0, u, dinv = _mlp_tc(x, W1, b1.reshape(1, HID), W2, b2.reshape(1, C),
                          deg_p.reshape(NC, NPAD).T)
    for _ in range(K - 1):
        s_p = _edge_sc(u, ridx, cidx, zeros2)
        u = _combine_tc(dinv, h0, u, s_p)
    s_p = _edge_sc(u, ridx, cidx, zeros2)
    return _final_tc(dinv, h0, u, s_p)


# speedup vs baseline: 24.5596x; 1.4073x over previous
import functools

import jax
import jax.numpy as jnp
from jax import lax
from jax.experimental import pallas as pl
from jax.experimental.pallas import tpu as pltpu
from jax.experimental.pallas import tpu_sc as plsc

N = 50000
E = 1600000
F_IN = 128
HID = 64
C = 40
K = 10
ALPHA = 0.1

NC = 2
NS = 16
CR = C // NC
CF = 32
B = 128
CH = 16
OUTER = 49
NCHUNK = CH * OUTER
E_PAD = NS * NCHUNK * B
CHD = 8
HALF = NCHUNK // 2
OUTER_D = HALF // CHD
NBUF = 6
AHEAD = 4
NPAD = 50048
ZR = NPAD // NS

_mesh = plsc.VectorSubcoreMesh(core_axis_name="c", subcore_axis_name="s")


@functools.partial(
    pl.kernel,
    mesh=_mesh,
    out_type=jax.ShapeDtypeStruct((NC * NPAD,), jnp.float32),
    scratch_types=[
        pltpu.VMEM((CHD, B), jnp.int32),
        pltpu.VMEM((B,), jnp.float32),
        pltpu.VMEM_SHARED((NPAD,), jnp.float32),
        pltpu.SemaphoreType.DMA,
    ],
    compiler_params=pltpu.CompilerParams(use_tc_tiling_on_sc=False),
)
def _deg_sc(cidx_hbm, zeros1_hbm, out_hbm, cidx_v, ones_v, acc, sem):
    c = lax.axis_index("c")
    s = lax.axis_index("s")
    pltpu.sync_copy(zeros1_hbm, acc.at[pl.ds(s * ZR, ZR)])
    for i in range(B // 16):
        ones_v[pl.ds(i * 16, 16)] = jnp.full((16,), 1.0, jnp.float32)
    plsc.subcore_barrier()

    def body(k, carry):
        pltpu.sync_copy(cidx_hbm.at[s, pl.ds(c * HALF + k * CHD, CHD)], cidx_v)
        for j in range(CHD):
            pltpu.sync_copy(ones_v, acc.at[cidx_v.at[j]], add=True)
        return carry

    lax.fori_loop(0, OUTER_D, body, 0)
    plsc.subcore_barrier()
    pltpu.sync_copy(acc.at[pl.ds(s * ZR, ZR)],
                    out_hbm.at[pl.ds(c * NPAD + s * ZR, ZR)])


@functools.partial(
    pl.kernel,
    mesh=_mesh,
    out_type=jax.ShapeDtypeStruct((NC, NPAD, CF), jnp.float32),
    scratch_types=[
        pltpu.VMEM((CH, B), jnp.int32),
        pltpu.VMEM((CH, B), jnp.int32),
        pltpu.VMEM_SHARED((NPAD, CF), jnp.float32),
    ] + [pltpu.VMEM((B, CF), jnp.float32) for _ in range(NBUF)]
      + [pltpu.SemaphoreType.DMA for _ in range(2 * NBUF)],
    compiler_params=pltpu.CompilerParams(use_tc_tiling_on_sc=False),
)
def _edge_sc(u2_hbm, ridx_hbm, cidx_hbm, zeros2_hbm, out_hbm,
             ridx_v, cidx_v, acc, *bufs):
    rows = bufs[:NBUF]
    gsem = bufs[NBUF:2 * NBUF]
    ssem = bufs[2 * NBUF:3 * NBUF]
    c = lax.axis_index("c")
    s = lax.axis_index("s")
    pltpu.sync_copy(zeros2_hbm, acc.at[pl.ds(s * ZR, ZR)])
    plsc.subcore_barrier()
    uc = u2_hbm.at[c]

    def body(m, carry):
        pltpu.sync_copy(ridx_hbm.at[s, pl.ds(m * CH, CH)], ridx_v)
        pltpu.sync_copy(cidx_hbm.at[s, pl.ds(m * CH, CH)], cidx_v)
        g = [None] * CH
        sc = [None] * CH
        for j in range(AHEAD):
            g[j] = pltpu.async_copy(uc.at[ridx_v.at[j]], rows[j], gsem[j])
        for j in range(CH):
            if j + AHEAD < CH:
                if j + AHEAD - NBUF >= 0:
                    sc[j + AHEAD - NBUF].wait()
                bj = (j + AHEAD) % NBUF
                g[j + AHEAD] = pltpu.async_copy(
                    uc.at[ridx_v.at[j + AHEAD]], rows[bj], gsem[bj])
            g[j].wait()
            sc[j] = pltpu.async_copy(
                rows[j % NBUF], acc.at[cidx_v.at[j]], ssem[j % NBUF],
                add=True)
        for j in range(CH - NBUF, CH):
            sc[j].wait()
        return carry

    lax.fori_loop(0, OUTER, body, 0)
    plsc.subcore_barrier()
    pltpu.sync_copy(acc.at[pl.ds(s * ZR, ZR)],
                    out_hbm.at[c, pl.ds(s * ZR, ZR)])


BN = 2000
GRID = N // BN


def _mlp_body(x_ref, w1_ref, b1_ref, w2_ref, b2_ref, deg_ref,
              h0_ref, u_ref, dinv_ref):
    h1 = jnp.maximum(
        jnp.dot(x_ref[...], w1_ref[...], preferred_element_type=jnp.float32)
        + b1_ref[...], 0.0)
    h0 = (jnp.dot(h1, w2_ref[...], preferred_element_type=jnp.float32)
          + b2_ref[...])
    deg = deg_ref[:, 0] + deg_ref[:, 1] + 1.0
    dinv = lax.rsqrt(deg)[:, None]
    h0_ref[...] = h0
    u = dinv * h0
    zpad = jnp.zeros((u.shape[0], CF - CR), jnp.float32)
    u_ref[0] = jnp.concatenate([u[:, :CR], zpad], axis=1)
    u_ref[1] = jnp.concatenate([u[:, CR:], zpad], axis=1)
    dinv_ref[...] = dinv


_mlp_tc = pl.pallas_call(
    _mlp_body,
    grid=(GRID,),
    in_specs=[
        pl.BlockSpec((BN, F_IN), lambda i: (i, 0)),
        pl.BlockSpec((F_IN, HID), lambda i: (0, 0)),
        pl.BlockSpec((1, HID), lambda i: (0, 0)),
        pl.BlockSpec((HID, C), lambda i: (0, 0)),
        pl.BlockSpec((1, C), lambda i: (0, 0)),
        pl.BlockSpec((BN, NC), lambda i: (i, 0)),
    ],
    out_specs=[
        pl.BlockSpec((BN, C), lambda i: (i, 0)),
        pl.BlockSpec((NC, BN, CF), lambda i: (0, i, 0)),
        pl.BlockSpec((BN, 1), lambda i: (i, 0)),
    ],
    out_shape=[
        jax.ShapeDtypeStruct((N, C), jnp.float32),
        jax.ShapeDtypeStruct((NC, N, CF), jnp.float32),
        jax.ShapeDtypeStruct((N, 1), jnp.float32),
    ],
)


def _combine_body(dinv_ref, h0_ref, u_ref, s_ref, out_ref):
    dinv = dinv_ref[...]
    for c in range(NC):
        agg = dinv * (s_ref[c][:, :CR] + u_ref[c][:, :CR])
        h0c = h0_ref[:, c * CR:(c + 1) * CR]
        nxt = dinv * ((1.0 - ALPHA) * agg + ALPHA * h0c)
        zpad = jnp.zeros((nxt.shape[0], CF - CR), jnp.float32)
        out_ref[c] = jnp.concatenate([nxt, zpad], axis=1)


def _final_body(dinv_ref, h0_ref, u_ref, s_ref, out_ref):
    dinv = dinv_ref[...]
    halves = []
    for c in range(NC):
        agg = dinv * (s_ref[c][:, :CR] + u_ref[c][:, :CR])
        h0c = h0_ref[:, c * CR:(c + 1) * CR]
        halves.append((1.0 - ALPHA) * agg + ALPHA * h0c)
    h = jnp.concatenate(halves, axis=1)
    m = jnp.max(h, axis=1, keepdims=True)
    lse = jnp.log(jnp.sum(jnp.exp(h - m), axis=1, keepdims=True)) + m
    out_ref[...] = h - lse


def _make_tc(body, out_spec, out_shape):
    return pl.pallas_call(
        body,
        grid=(GRID,),
        in_specs=[
            pl.BlockSpec((BN, 1), lambda i: (i, 0)),
            pl.BlockSpec((BN, C), lambda i: (i, 0)),
            pl.BlockSpec((NC, BN, CF), lambda i: (0, i, 0)),
            pl.BlockSpec((NC, BN, CF), lambda i: (0, i, 0)),
        ],
        out_specs=out_spec,
        out_shape=out_shape,
    )


_combine_tc = _make_tc(
    _combine_body,
    pl.BlockSpec((NC, BN, CF), lambda i: (0, i, 0)),
    jax.ShapeDtypeStruct((NC, N, CF), jnp.float32),
)
_final_tc = _make_tc(
    _final_body,
    pl.BlockSpec((BN, C), lambda i: (i, 0)),
    jax.ShapeDtypeStruct((N, C), jnp.float32),
)


def kernel(x, edge_index, y, W1, b1, W2, b2):
    del y
    row = edge_index[0].astype(jnp.int32)
    col = edge_index[1].astype(jnp.int32)
    pad = E_PAD - E
    ridx = jnp.concatenate([row, jnp.zeros((pad,), jnp.int32)])
    cidx = jnp.concatenate([col, jnp.full((pad,), N, jnp.int32)])
    ridx = ridx.reshape(NS, NCHUNK, B)
    cidx = cidx.reshape(NS, NCHUNK, B)
    zeros1 = jnp.zeros((ZR,), jnp.float32)
    zeros2 = jnp.zeros((ZR, CF), jnp.float32)

    deg_p = _deg_sc(cidx, zeros1)
    h0, u, dinv = _mlp_tc(x, W1, b1.reshape(1, HID), W2, b2.reshape(1, C),
                          deg_p.reshape(NC, NPAD).T)
    for _ in range(K - 1):
        s_p = _edge_sc(u, ridx, cidx, zeros2)
        u = _combine_tc(dinv, h0, u, s_p)
    s_p = _edge_sc(u, ridx, cidx, zeros2)
    return _final_tc(dinv, h0, u, s_p)


# speedup vs baseline: 24.6613x; 1.0041x over previous
import functools

import jax
import jax.numpy as jnp
from jax import lax
from jax.experimental import pallas as pl
from jax.experimental.pallas import tpu as pltpu
from jax.experimental.pallas import tpu_sc as plsc

N = 50000
E = 1600000
F_IN = 128
HID = 64
C = 40
K = 10
ALPHA = 0.1

NC = 2
NS = 16
CR = C // NC
CF = 32
B = 128
CH = 16
OUTER = 49
NCHUNK = CH * OUTER
E_PAD = NS * NCHUNK * B
CHD = 8
HALF = NCHUNK // 2
OUTER_D = HALF // CHD
NBUF = 6
AHEAD = 4
NPAD = 50048
ZR = NPAD // NS

_mesh = plsc.VectorSubcoreMesh(core_axis_name="c", subcore_axis_name="s")


@functools.partial(
    pl.kernel,
    mesh=_mesh,
    out_type=jax.ShapeDtypeStruct((NC * NPAD,), jnp.float32),
    scratch_types=[
        pltpu.VMEM((CHD, B), jnp.int32),
        pltpu.VMEM((B,), jnp.float32),
        pltpu.VMEM_SHARED((NPAD,), jnp.float32),
        pltpu.SemaphoreType.DMA,
        pltpu.SemaphoreType.DMA,
        pltpu.SemaphoreType.DMA,
        pltpu.SemaphoreType.DMA,
    ],
    compiler_params=pltpu.CompilerParams(use_tc_tiling_on_sc=False),
)
def _deg_sc(cidx_hbm, zeros1_hbm, out_hbm, cidx_v, ones_v, acc, *dsems):
    c = lax.axis_index("c")
    s = lax.axis_index("s")
    pltpu.sync_copy(zeros1_hbm, acc.at[pl.ds(s * ZR, ZR)])
    for i in range(B // 16):
        ones_v[pl.ds(i * 16, 16)] = jnp.full((16,), 1.0, jnp.float32)
    plsc.subcore_barrier()

    def body(k, carry):
        pltpu.sync_copy(cidx_hbm.at[s, pl.ds(c * HALF + k * CHD, CHD)], cidx_v)
        scs = [pltpu.async_copy(ones_v, acc.at[cidx_v.at[j]],
                                dsems[j % 4], add=True)
               for j in range(CHD)]
        for sc_ in scs:
            sc_.wait()
        return carry

    lax.fori_loop(0, OUTER_D, body, 0)
    plsc.subcore_barrier()
    pltpu.sync_copy(acc.at[pl.ds(s * ZR, ZR)],
                    out_hbm.at[pl.ds(c * NPAD + s * ZR, ZR)])


@functools.partial(
    pl.kernel,
    mesh=_mesh,
    out_type=jax.ShapeDtypeStruct((NC, NPAD, CF), jnp.float32),
    scratch_types=[
        pltpu.VMEM((CH, B), jnp.int32),
        pltpu.VMEM((CH, B), jnp.int32),
        pltpu.VMEM_SHARED((NPAD, CF), jnp.float32),
    ] + [pltpu.VMEM((B, CF), jnp.float32) for _ in range(NBUF)]
      + [pltpu.SemaphoreType.DMA for _ in range(2 * NBUF)],
    compiler_params=pltpu.CompilerParams(use_tc_tiling_on_sc=False),
)
def _edge_sc(u2_hbm, ridx_hbm, cidx_hbm, zeros2_hbm, out_hbm,
             ridx_v, cidx_v, acc, *bufs):
    rows = bufs[:NBUF]
    gsem = bufs[NBUF:2 * NBUF]
    ssem = bufs[2 * NBUF:3 * NBUF]
    c = lax.axis_index("c")
    s = lax.axis_index("s")
    pltpu.sync_copy(zeros2_hbm, acc.at[pl.ds(s * ZR, ZR)])
    plsc.subcore_barrier()
    uc = u2_hbm.at[c]

    def body(m, carry):
        pltpu.sync_copy(ridx_hbm.at[s, pl.ds(m * CH, CH)], ridx_v)
        pltpu.sync_copy(cidx_hbm.at[s, pl.ds(m * CH, CH)], cidx_v)
        g = [None] * CH
        sc = [None] * CH
        for j in range(AHEAD):
            g[j] = pltpu.async_copy(uc.at[ridx_v.at[j]], rows[j], gsem[j])
        for j in range(CH):
            if j + AHEAD < CH:
                if j + AHEAD - NBUF >= 0:
                    sc[j + AHEAD - NBUF].wait()
                bj = (j + AHEAD) % NBUF
                g[j + AHEAD] = pltpu.async_copy(
                    uc.at[ridx_v.at[j + AHEAD]], rows[bj], gsem[bj])
            g[j].wait()
            sc[j] = pltpu.async_copy(
                rows[j % NBUF], acc.at[cidx_v.at[j]], ssem[j % NBUF],
                add=True)
        for j in range(CH - NBUF, CH):
            sc[j].wait()
        return carry

    lax.fori_loop(0, OUTER, body, 0)
    plsc.subcore_barrier()
    pltpu.sync_copy(acc.at[pl.ds(s * ZR, ZR)],
                    out_hbm.at[c, pl.ds(s * ZR, ZR)])


BN = 2000
GRID = N // BN


def _mlp_body(x_ref, w1_ref, b1_ref, w2_ref, b2_ref, deg_ref,
              h0_ref, u_ref, dinv_ref):
    h1 = jnp.maximum(
        jnp.dot(x_ref[...], w1_ref[...], preferred_element_type=jnp.float32)
        + b1_ref[...], 0.0)
    h0 = (jnp.dot(h1, w2_ref[...], preferred_element_type=jnp.float32)
          + b2_ref[...])
    deg = deg_ref[:, 0] + deg_ref[:, 1] + 1.0
    dinv = lax.rsqrt(deg)[:, None]
    h0_ref[...] = h0
    u = dinv * h0
    zpad = jnp.zeros((u.shape[0], CF - CR), jnp.float32)
    u_ref[0] = jnp.concatenate([u[:, :CR], zpad], axis=1)
    u_ref[1] = jnp.concatenate([u[:, CR:], zpad], axis=1)
    dinv_ref[...] = dinv


_mlp_tc = pl.pallas_call(
    _mlp_body,
    grid=(GRID,),
    in_specs=[
        pl.BlockSpec((BN, F_IN), lambda i: (i, 0)),
        pl.BlockSpec((F_IN, HID), lambda i: (0, 0)),
        pl.BlockSpec((1, HID), lambda i: (0, 0)),
        pl.BlockSpec((HID, C), lambda i: (0, 0)),
        pl.BlockSpec((1, C), lambda i: (0, 0)),
        pl.BlockSpec((BN, NC), lambda i: (i, 0)),
    ],
    out_specs=[
        pl.BlockSpec((BN, C), lambda i: (i, 0)),
        pl.BlockSpec((NC, BN, CF), lambda i: (0, i, 0)),
        pl.BlockSpec((BN, 1), lambda i: (i, 0)),
    ],
    out_shape=[
        jax.ShapeDtypeStruct((N, C), jnp.float32),
        jax.ShapeDtypeStruct((NC, N, CF), jnp.float32),
        jax.ShapeDtypeStruct((N, 1), jnp.float32),
    ],
)


def _combine_body(dinv_ref, h0_ref, u_ref, s_ref, out_ref):
    dinv = dinv_ref[...]
    for c in range(NC):
        agg = dinv * (s_ref[c][:, :CR] + u_ref[c][:, :CR])
        h0c = h0_ref[:, c * CR:(c + 1) * CR]
        nxt = dinv * ((1.0 - ALPHA) * agg + ALPHA * h0c)
        zpad = jnp.zeros((nxt.shape[0], CF - CR), jnp.float32)
        out_ref[c] = jnp.concatenate([nxt, zpad], axis=1)


def _final_body(dinv_ref, h0_ref, u_ref, s_ref, out_ref):
    dinv = dinv_ref[...]
    halves = []
    for c in range(NC):
        agg = dinv * (s_ref[c][:, :CR] + u_ref[c][:, :CR])
        h0c = h0_ref[:, c * CR:(c + 1) * CR]
        halves.append((1.0 - ALPHA) * agg + ALPHA * h0c)
    h = jnp.concatenate(halves, axis=1)
    m = jnp.max(h, axis=1, keepdims=True)
    lse = jnp.log(jnp.sum(jnp.exp(h - m), axis=1, keepdims=True)) + m
    out_ref[...] = h - lse


def _make_tc(body, out_spec, out_shape):
    return pl.pallas_call(
        body,
        grid=(GRID,),
        in_specs=[
            pl.BlockSpec((BN, 1), lambda i: (i, 0)),
            pl.BlockSpec((BN, C), lambda i: (i, 0)),
            pl.BlockSpec((NC, BN, CF), lambda i: (0, i, 0)),
            pl.BlockSpec((NC, BN, CF), lambda i: (0, i, 0)),
        ],
        out_specs=out_spec,
        out_shape=out_shape,
    )


_combine_tc = _make_tc(
    _combine_body,
    pl.BlockSpec((NC, BN, CF), lambda i: (0, i, 0)),
    jax.ShapeDtypeStruct((NC, N, CF), jnp.float32),
)
_final_tc = _make_tc(
    _final_body,
    pl.BlockSpec((BN, C), lambda i: (i, 0)),
    jax.ShapeDtypeStruct((N, C), jnp.float32),
)


def kernel(x, edge_index, y, W1, b1, W2, b2):
    del y
    row = edge_index[0].astype(jnp.int32)
    col = edge_index[1].astype(jnp.int32)
    pad = E_PAD - E
    ridx = jnp.concatenate([row, jnp.zeros((pad,), jnp.int32)])
    cidx = jnp.concatenate([col, jnp.full((pad,), N, jnp.int32)])
    ridx = ridx.reshape(NS, NCHUNK, B)
    cidx = cidx.reshape(NS, NCHUNK, B)
    zeros1 = jnp.zeros((ZR,), jnp.float32)
    zeros2 = jnp.zeros((ZR, CF), jnp.float32)

    deg_p = _deg_sc(cidx, zeros1)
    h0, u, dinv = _mlp_tc(x, W1, b1.reshape(1, HID), W2, b2.reshape(1, C),
                          deg_p.reshape(NC, NPAD).T)
    for _ in range(K - 1):
        s_p = _edge_sc(u, ridx, cidx, zeros2)
        u = _combine_tc(dinv, h0, u, s_p)
    s_p = _edge_sc(u, ridx, cidx, zeros2)
    return _final_tc(dinv, h0, u, s_p)
